# baseline (speedup 1.0000x reference)
import jax
import jax.numpy as jnp
from jax.experimental import pallas as pl
from jax.experimental.pallas import tpu as pltpu

_C = 1000
_RATIO = 3
_ROWS_PER_BLOCK = 1024
_LOG2E = 1.4426950408889634
_LN2 = 0.6931471805599453


def _softplus(x):
    u = jax.lax.exp2(jnp.abs(x) * jnp.float32(-_LOG2E))
    return jnp.maximum(x, 0.0) + jnp.log1p(u)


def _block_kernel(pred_ref, target_ref, num_ref, den_ref, acc_ref):
    x = pred_ref[...]
    y = target_ref[...]
    s = _softplus(x)
    contrib = s - y * x
    i = pl.program_id(0)

    @pl.when(i == 0)
    def _init():
        num_ref[...] = jnp.zeros((1, 1), jnp.float32)
        den_ref[...] = jnp.zeros((1, 1), jnp.float32)
        acc_ref[...] = jnp.zeros_like(acc_ref)

    acc_ref[...] += contrib
    pos = jnp.sum(y, axis=1)
    den_ref[...] += jnp.sum(pos).reshape(1, 1)
    d_f = jnp.maximum(_C - (_RATIO + 1.0) * pos, 0.0)

    @pl.when(jnp.any(d_f > 0.0))
    def _rare_correction():
        b = jax.lax.bitcast_convert_type(x, jnp.int32)
        ikey = b ^ ((b >> 31) & jnp.int32(0x7FFFFFFF))
        ikey = jnp.where(y > 0.5, jnp.int32(0x7FFFFFFF), ikey)
        d = d_f.astype(jnp.int32)
        cnt_neg = jnp.sum((ikey < 0).astype(jnp.int32), axis=1)
        t0 = jnp.where(cnt_neg >= d, jnp.int32(-2147483648), jnp.int32(0))

        def body(j, t):
            cand = t + (jnp.int32(1) << (30 - j))
            cnt = jnp.sum((ikey < cand[:, None]).astype(jnp.int32), axis=1)
            return jnp.where(cnt < d, cand, t)

        t = jax.lax.fori_loop(0, 31, body, t0)
        below = ikey < t[:, None]
        cnt_lt = jnp.sum(below.astype(jnp.int32), axis=1)
        sum_below = jnp.sum(jnp.where(below, s, 0.0), axis=1)
        bv = t ^ ((t >> 31) & jnp.int32(0x7FFFFFFF))
        sv = _softplus(jax.lax.bitcast_convert_type(bv, jnp.float32))
        corr = sum_below + (d - cnt_lt).astype(jnp.float32) * sv
        corr = jnp.where(d > 0, corr, 0.0)
        num_ref[...] += -jnp.sum(corr).reshape(1, 1)

    @pl.when(i == pl.num_programs(0) - 1)
    def _finalize():
        num_ref[...] += jnp.sum(acc_ref[...]).reshape(1, 1)


def kernel(pred, target):
    n, c = pred.shape
    r = _ROWS_PER_BLOCK
    num, den = pl.pallas_call(
        _block_kernel,
        grid=(n // r,),
        in_specs=[
            pl.BlockSpec((r, c), lambda i: (i, 0)),
            pl.BlockSpec((r, c), lambda i: (i, 0)),
        ],
        out_specs=[
            pl.BlockSpec((1, 1), lambda i: (0, 0)),
            pl.BlockSpec((1, 1), lambda i: (0, 0)),
        ],
        out_shape=[
            jax.ShapeDtypeStruct((1, 1), jnp.float32),
            jax.ShapeDtypeStruct((1, 1), jnp.float32),
        ],
        scratch_shapes=[pltpu.VMEM((r, c), jnp.float32)],
    )(pred, target)
    return (num[0, 0] / c) / den[0, 0]


# speedup vs baseline: 1.0133x; 1.0133x over previous
import jax
import jax.numpy as jnp
from jax.experimental import pallas as pl
from jax.experimental.pallas import tpu as pltpu

_C = 1000
_RATIO = 3
_ROWS_PER_BLOCK = 512
_LOG2E = 1.4426950408889634
_LN2 = 0.6931471805599453


def _softplus(x):
    u = jax.lax.exp2(jnp.abs(x) * jnp.float32(-_LOG2E))
    return jnp.maximum(x, 0.0) + jnp.log1p(u)


def _block_kernel(pred_ref, target_ref, num_ref, den_ref, acc_ref):
    x = pred_ref[...]
    y = target_ref[...]
    s = _softplus(x)
    contrib = s - y * x
    i = pl.program_id(0)

    @pl.when(i == 0)
    def _init():
        num_ref[...] = jnp.zeros((1, 1), jnp.float32)
        den_ref[...] = jnp.zeros((1, 1), jnp.float32)
        acc_ref[...] = jnp.zeros_like(acc_ref)

    acc_ref[...] += contrib
    pos = jnp.sum(y, axis=1)
    den_ref[...] += jnp.sum(pos).reshape(1, 1)
    d_f = jnp.maximum(_C - (_RATIO + 1.0) * pos, 0.0)

    @pl.when(jnp.any(d_f > 0.0))
    def _rare_correction():
        b = jax.lax.bitcast_convert_type(x, jnp.int32)
        ikey = b ^ ((b >> 31) & jnp.int32(0x7FFFFFFF))
        ikey = jnp.where(y > 0.5, jnp.int32(0x7FFFFFFF), ikey)
        d = d_f.astype(jnp.int32)
        cnt_neg = jnp.sum((ikey < 0).astype(jnp.int32), axis=1)
        t0 = jnp.where(cnt_neg >= d, jnp.int32(-2147483648), jnp.int32(0))

        def body(j, t):
            cand = t + (jnp.int32(1) << (30 - j))
            cnt = jnp.sum((ikey < cand[:, None]).astype(jnp.int32), axis=1)
            return jnp.where(cnt < d, cand, t)

        t = jax.lax.fori_loop(0, 31, body, t0)
        below = ikey < t[:, None]
        cnt_lt = jnp.sum(below.astype(jnp.int32), axis=1)
        sum_below = jnp.sum(jnp.where(below, s, 0.0), axis=1)
        bv = t ^ ((t >> 31) & jnp.int32(0x7FFFFFFF))
        sv = _softplus(jax.lax.bitcast_convert_type(bv, jnp.float32))
        corr = sum_below + (d - cnt_lt).astype(jnp.float32) * sv
        corr = jnp.where(d > 0, corr, 0.0)
        num_ref[...] += -jnp.sum(corr).reshape(1, 1)

    @pl.when(i == pl.num_programs(0) - 1)
    def _finalize():
        num_ref[...] += jnp.sum(acc_ref[...]).reshape(1, 1)


def kernel(pred, target):
    n, c = pred.shape
    r = _ROWS_PER_BLOCK
    num, den = pl.pallas_call(
        _block_kernel,
        grid=(n // r,),
        in_specs=[
            pl.BlockSpec((r, c), lambda i: (i, 0)),
            pl.BlockSpec((r, c), lambda i: (i, 0)),
        ],
        out_specs=[
            pl.BlockSpec((1, 1), lambda i: (0, 0)),
            pl.BlockSpec((1, 1), lambda i: (0, 0)),
        ],
        out_shape=[
            jax.ShapeDtypeStruct((1, 1), jnp.float32),
            jax.ShapeDtypeStruct((1, 1), jnp.float32),
        ],
        scratch_shapes=[pltpu.VMEM((r, c), jnp.float32)],
    )(pred, target)
    return (num[0, 0] / c) / den[0, 0]


# speedup vs baseline: 1.0384x; 1.0248x over previous
import functools

import jax
import jax.numpy as jnp
from jax.experimental import pallas as pl
from jax.experimental.pallas import tpu as pltpu

_C = 1000
_RATIO = 3
_CHUNK = 512
_DEPTH = 4
_LOG2E = 1.4426950408889634


def _softplus(x):
    u = jax.lax.exp2(jnp.abs(x) * jnp.float32(-_LOG2E))
    return jnp.maximum(x, 0.0) + jnp.log1p(u)


def _dma(hbm, buf, sems, slot, chunk):
    return pltpu.make_async_copy(
        hbm.at[pl.ds(chunk * _CHUNK, _CHUNK), :], buf.at[slot], sems.at[slot]
    )


def _block_kernel(pred_hbm, target_hbm, num_ref, den_ref, bufx, bufy, semx, semy,
                  *, nch):
    i = pl.program_id(0)

    @pl.when(i == 0)
    def _prologue():
        num_ref[...] = jnp.zeros((1, 1), jnp.float32)
        den_ref[...] = jnp.zeros((1, 1), jnp.float32)
        for b in range(min(_DEPTH - 1, nch)):
            _dma(pred_hbm, bufx, semx, b, b).start()
            _dma(target_hbm, bufy, semy, b, b).start()

    nxt = i + _DEPTH - 1

    @pl.when(nxt < nch)
    def _issue():
        slot = jax.lax.rem(nxt, _DEPTH)
        _dma(pred_hbm, bufx, semx, slot, nxt).start()
        _dma(target_hbm, bufy, semy, slot, nxt).start()

    slot = jax.lax.rem(i, _DEPTH)
    _dma(pred_hbm, bufx, semx, slot, i).wait()
    _dma(target_hbm, bufy, semy, slot, i).wait()
    x = bufx[slot]
    y = bufy[slot]

    s = _softplus(x)
    contrib = s - y * x
    pos = jnp.sum(y, axis=1)
    num_ref[...] += jnp.sum(contrib).reshape(1, 1)
    den_ref[...] += jnp.sum(pos).reshape(1, 1)
    d_f = jnp.maximum(_C - (_RATIO + 1.0) * pos, 0.0)

    @pl.when(jnp.any(d_f > 0.0))
    def _rare_correction():
        b = jax.lax.bitcast_convert_type(x, jnp.int32)
        ikey = b ^ ((b >> 31) & jnp.int32(0x7FFFFFFF))
        ikey = jnp.where(y > 0.5, jnp.int32(0x7FFFFFFF), ikey)
        d = d_f.astype(jnp.int32)
        cnt_neg = jnp.sum((ikey < 0).astype(jnp.int32), axis=1)
        t0 = jnp.where(cnt_neg >= d, jnp.int32(-2147483648), jnp.int32(0))

        def body(j, t):
            cand = t + (jnp.int32(1) << (30 - j))
            cnt = jnp.sum((ikey < cand[:, None]).astype(jnp.int32), axis=1)
            return jnp.where(cnt < d, cand, t)

        t = jax.lax.fori_loop(0, 31, body, t0)
        below = ikey < t[:, None]
        cnt_lt = jnp.sum(below.astype(jnp.int32), axis=1)
        sum_below = jnp.sum(jnp.where(below, s, 0.0), axis=1)
        bv = t ^ ((t >> 31) & jnp.int32(0x7FFFFFFF))
        sv = _softplus(jax.lax.bitcast_convert_type(bv, jnp.float32))
        corr = sum_below + (d - cnt_lt).astype(jnp.float32) * sv
        corr = jnp.where(d > 0, corr, 0.0)
        num_ref[...] += -jnp.sum(corr).reshape(1, 1)


def kernel(pred, target):
    n, c = pred.shape
    num, den = pl.pallas_call(
        functools.partial(_block_kernel, nch=n // _CHUNK),
        grid=(n // _CHUNK,),
        in_specs=[
            pl.BlockSpec(memory_space=pl.ANY),
            pl.BlockSpec(memory_space=pl.ANY),
        ],
        out_specs=[
            pl.BlockSpec((1, 1), lambda i: (0, 0)),
            pl.BlockSpec((1, 1), lambda i: (0, 0)),
        ],
        out_shape=[
            jax.ShapeDtypeStruct((1, 1), jnp.float32),
            jax.ShapeDtypeStruct((1, 1), jnp.float32),
        ],
        scratch_shapes=[
            pltpu.VMEM((_DEPTH, _CHUNK, c), jnp.float32),
            pltpu.VMEM((_DEPTH, _CHUNK, c), jnp.float32),
            pltpu.SemaphoreType.DMA((_DEPTH,)),
            pltpu.SemaphoreType.DMA((_DEPTH,)),
        ],
    )(pred, target)
    return (num[0, 0] / c) / den[0, 0]
